# baseline (speedup 1.0000x reference)
import functools

import jax
import jax.numpy as jnp
from jax import lax
from jax.experimental import pallas as pl
from jax.experimental.pallas import tpu as pltpu
from jax.experimental.pallas import tpu_sc as plsc

N = 10000
E = 640000
NUM_REL = 4
NUM_BASES = 2
HIDDEN = 64
OUT = 64

NW = 32
TPW = 20480
EP = NW * TPW
B = 1024
NCH = TPW // B
NG = B // 16
KD = B // 128

ACC0_ROWS = 10112
DEG_ROWS = 40960
DEG_STRIPE = DEG_ROWS // 16
A0_STRIPE = ACC0_ROWS // 16

_NBLK = 1000

_HI = jax.lax.Precision.HIGHEST
_mesh = plsc.VectorSubcoreMesh(core_axis_name="c", subcore_axis_name="s")
_SC_PARAMS = pltpu.CompilerParams(needs_layout_passes=False,
                                  use_tc_tiling_on_sc=False)


DEG_WORDS = 40016


def _deg_body(dst_hbm, t_hbm, outd, dstv, tv, degv):
    c = lax.axis_index("c")
    s = lax.axis_index("s")
    wid = c * 16 + s

    lanes = lax.iota(jnp.int32, 16)
    zero16 = jnp.zeros((16,), jnp.float32)
    ones16 = jnp.ones((16,), jnp.float32)

    def zdeg(g, _):
        degv[pl.ds(g * 16, 16)] = zero16
        return 0

    lax.fori_loop(0, DEG_WORDS // 16, zdeg, 0)

    def chunk(ci, _):
        base = wid * TPW + ci * B
        pltpu.sync_copy(dst_hbm.at[pl.ds(base, B)], dstv)
        pltpu.sync_copy(t_hbm.at[pl.ds(base, B)], tv)

        def grp(g, _):
            d16 = dstv[pl.ds(g * 16, 16)]
            t16 = tv[pl.ds(g * 16, 16)]
            gi = d16 * 4 + t16
            for j in range(16):
                plsc.addupdate_scatter(degv, [gi], ones16, mask=lanes == j)
            return 0

        lax.fori_loop(0, NG, grp, 0)
        return 0

    lax.fori_loop(0, NCH, chunk, 0)
    pltpu.sync_copy(degv, outd.at[pl.ds(wid * DEG_WORDS, DEG_WORDS)])


def _deg_counts(dstp, tp):
    f = pl.kernel(
        _deg_body,
        out_type=jax.ShapeDtypeStruct((NW * DEG_WORDS,), jnp.float32),
        mesh=_mesh,
        compiler_params=_SC_PARAMS,
        scratch_types=[
            pltpu.VMEM((B,), jnp.int32),
            pltpu.VMEM((B,), jnp.int32),
            pltpu.VMEM((DEG_WORDS,), jnp.float32),
        ],
    )
    return f(dstp, tp)


HALF = N // 2
ACC_ROWS = 5120
HW = 32
CAP = TPW + 16


def _part_body(src_hbm, dst_hbm, t_hbm, gi_out, di_out, cnt_out,
               srcv, dstv, tv, giA, diA, giB, diB, cbuf, sem_l):
    c = lax.axis_index("c")
    s = lax.axis_index("s")
    wid = c * 16 + s
    lanes = lax.iota(jnp.int32, 16)

    zero16 = jnp.zeros((16,), jnp.int32)
    padA = zero16 + 4 * HALF
    padB = zero16 + 4 * N

    def pre(g, _):
        giA[pl.ds(g * 16, 16)] = zero16
        diA[pl.ds(g * 16, 16)] = padA
        giB[pl.ds(g * 16, 16)] = zero16
        diB[pl.ds(g * 16, 16)] = padB
        return 0

    lax.fori_loop(0, CAP // 16, pre, 0)

    def chunk(ci, offs):
        base = wid * TPW + ci * B
        dls = [pltpu.async_copy(src_hbm.at[pl.ds(base, B)], srcv, sem_l),
               pltpu.async_copy(dst_hbm.at[pl.ds(base, B)], dstv, sem_l),
               pltpu.async_copy(t_hbm.at[pl.ds(base, B)], tv, sem_l)]
        for d in dls:
            d.wait()

        def grp(g, offs2):
            offa, offb = offs2
            s16 = srcv[pl.ds(g * 16, 16)]
            d16 = dstv[pl.ds(g * 16, 16)]
            t16 = tv[pl.ds(g * 16, 16)]
            gi = t16 * N + s16
            di = d16 * 4 + t16
            ma = d16 < HALF
            mb = (d16 >= HALF) & (d16 < N)
            plsc.store_compressed(giA.at[pl.ds(offa, 16)], gi, mask=ma)
            plsc.store_compressed(diA.at[pl.ds(offa, 16)], di, mask=ma)
            plsc.store_compressed(giB.at[pl.ds(offb, 16)], gi, mask=mb)
            plsc.store_compressed(diB.at[pl.ds(offb, 16)], di, mask=mb)
            na = jnp.sum(ma.astype(jnp.int32))
            nb = jnp.sum(mb.astype(jnp.int32))
            return (offa + na, offb + nb)

        return lax.fori_loop(0, NG, grp, offs)

    offa, offb = lax.fori_loop(0, NCH, chunk, (0, 0))

    cbuf[pl.ds(0, 16)] = jnp.where(lanes == 0, offa, 0)
    pltpu.sync_copy(cbuf, cnt_out.at[pl.ds(wid * 16, 16)])
    cbuf[pl.ds(0, 16)] = jnp.where(lanes == 0, offb, 0)
    pltpu.sync_copy(cbuf, cnt_out.at[pl.ds((32 + wid) * 16, 16)])

    pltpu.sync_copy(giA.at[pl.ds(0, TPW)], gi_out.at[pl.ds(wid * TPW, TPW)])
    pltpu.sync_copy(diA.at[pl.ds(0, TPW)], di_out.at[pl.ds(wid * TPW, TPW)])
    pltpu.sync_copy(giB.at[pl.ds(0, TPW)],
                    gi_out.at[pl.ds((32 + wid) * TPW, TPW)])
    pltpu.sync_copy(diB.at[pl.ds(0, TPW)],
                    di_out.at[pl.ds((32 + wid) * TPW, TPW)])


def _partition(srcp, dstp, tp):
    f = pl.kernel(
        _part_body,
        out_type=(
            jax.ShapeDtypeStruct((64 * TPW,), jnp.int32),
            jax.ShapeDtypeStruct((64 * TPW,), jnp.int32),
            jax.ShapeDtypeStruct((64 * 16,), jnp.int32),
        ),
        mesh=_mesh,
        compiler_params=_SC_PARAMS,
        scratch_types=[
            pltpu.VMEM((B,), jnp.int32),
            pltpu.VMEM((B,), jnp.int32),
            pltpu.VMEM((B,), jnp.int32),
            pltpu.VMEM((CAP,), jnp.int32),
            pltpu.VMEM((CAP,), jnp.int32),
            pltpu.VMEM((CAP,), jnp.int32),
            pltpu.VMEM((CAP,), jnp.int32),
            pltpu.VMEM((16,), jnp.int32),
            pltpu.SemaphoreType.DMA,
        ],
    )
    return f(srcp, dstp, tp)


def _edge_body(gi_hbm, di_hbm, cnt_hbm, table_hbm, recip_hbm, outp,
               gilv, dilv, sidx, coefv, rows, recipv, cbufv, acc,
               sem_l, *sems):
    sem_g, sem_s = sems[:KD], sems[KD:]
    c = lax.axis_index("c")
    s = lax.axis_index("s")
    lanes = lax.iota(jnp.int32, 16)
    lo = c * HALF

    pltpu.sync_copy(recip_hbm, recipv.at[pl.ds(0, 4 * N)])
    recipv[pl.ds(4 * N, 16)] = jnp.zeros((16,), jnp.float32)

    zero16 = jnp.zeros((16,), jnp.float32)

    def zrow(i, _):
        for k in range(HW // 16):
            rows[i, pl.ds(k * 16, 16)] = zero16
        return 0

    lax.fori_loop(0, B, zrow, 0)
    pltpu.sync_copy(rows.at[pl.ds(0, 320)], acc.at[pl.ds(s * 320, 320)])
    plsc.subcore_barrier()

    for k in range(2):
        q = c * 32 + 2 * s + k
        pltpu.sync_copy(cnt_hbm.at[pl.ds(q * 16, 16)], cbufv)
        cv = cbufv[pl.ds(0, 16)]
        cnt = jnp.sum(jnp.where(lanes == 0, cv, 0))
        nch = (cnt + B - 1) // B

        dpre = [pltpu.async_copy(gi_hbm.at[pl.ds(q * TPW, TPW)], gilv, sem_l),
                pltpu.async_copy(di_hbm.at[pl.ds(q * TPW, TPW)], dilv, sem_l)]
        for d in dpre:
            d.wait()

        def chunk(ci, _):
            cb = ci * B

            @plsc.parallel_loop(0, NG, 1, unroll=2)
            def grp(g):
                di16 = dilv[pl.ds(cb + g * 16, 16)]
                u16 = (di16 >> 2) - lo
                sidx[g // 8, pl.ds((g % 8) * 16, 16)] = jnp.where(
                    u16 >= HALF, HALF + lanes, u16)
                coefv[pl.ds(g * 16, 16)] = plsc.load_gather(recipv, [di16])

            dgs = [pltpu.async_copy(
                       table_hbm.at[gilv.at[pl.ds(cb + j * 128, 128)]],
                       rows.at[pl.ds(j * 128, 128)], sem_g[j])
                   for j in range(KD)]
            dss = []
            for j in range(KD):
                dgs[j].wait()

                @plsc.parallel_loop(0, 8, 1, unroll=2)
                def scale(g):
                    for jj in range(16):
                        e = j * 128 + g * 16 + jj
                        cof = plsc.load_gather(
                            coefv, [jnp.zeros((16,), jnp.int32) + e])
                        for kk in range(HW // 16):
                            rows[e, pl.ds(kk * 16, 16)] = (
                                rows[e, pl.ds(kk * 16, 16)] * cof)

                dss.append(pltpu.async_copy(rows.at[pl.ds(j * 128, 128)],
                                            acc.at[sidx.at[j]], sem_s[j],
                                            add=True))
            for d in dss:
                d.wait()
            return 0

        lax.fori_loop(0, nch, chunk, 0)

    plsc.subcore_barrier()

    pltpu.sync_copy(acc.at[pl.ds(s * 320, 320)],
                    outp.at[c, pl.ds(s * 320, 320)])


def _edge_pass(gip, dip, cnts, table, recip_flat):
    f = pl.kernel(
        _edge_body,
        out_type=jax.ShapeDtypeStruct((2, ACC_ROWS, HW), jnp.float32),
        mesh=_mesh,
        compiler_params=_SC_PARAMS,
        scratch_types=[
            pltpu.VMEM((TPW,), jnp.int32),
            pltpu.VMEM((TPW,), jnp.int32),
            pltpu.VMEM((KD, 128), jnp.int32),
            pltpu.VMEM((B,), jnp.float32),
            pltpu.VMEM((B, HW), jnp.float32),
            pltpu.VMEM((4 * N + 16,), jnp.float32),
            pltpu.VMEM((16,), jnp.int32),
            pltpu.VMEM_SHARED((ACC_ROWS, HW), jnp.float32),
            pltpu.SemaphoreType.DMA,
        ] + [pltpu.SemaphoreType.DMA] * (2 * KD),
    )
    return f(gip, dip, cnts, table, recip_flat)


def _prep_body(parts_ref, wb0f_ref, wb0id_ref, wr0_ref,
               recip_ref, bias_ref, t0_ref):
    d = jnp.sum(parts_ref[...], axis=0)
    recip_ref[...] = jnp.where(d > 0, 1.0 / jnp.maximum(d, 1.0), 0.0)
    ind = (d > 0).astype(jnp.float32)
    wr0 = wr0_ref[...]
    colsum = jnp.sum(wb0f_ref[...], axis=1)
    sm = jnp.dot(wr0, colsum, precision=_HI)
    bias_ref[...] = jnp.dot(ind, sm, precision=_HI)
    for r in range(NUM_REL):
        accv = wr0[r, 0] * wb0id_ref[0]
        for b in range(1, NUM_BASES):
            accv = accv + wr0[r, b] * wb0id_ref[b]
        t0_ref[r] = accv


def _prep(deg_parts, w_bases0, w_rel0):
    wb0f = w_bases0[:, :OUT, :]
    wb0id = w_bases0[:, OUT:, :]
    grid = (N // _NBLK,)
    return pl.pallas_call(
        _prep_body,
        grid=grid,
        in_specs=[
            pl.BlockSpec((NW, _NBLK, NUM_REL), lambda j: (0, j, 0)),
            pl.BlockSpec((NUM_BASES, OUT, HIDDEN), lambda j: (0, 0, 0)),
            pl.BlockSpec((NUM_BASES, _NBLK, HIDDEN), lambda j: (0, j, 0)),
            pl.BlockSpec((NUM_REL, NUM_BASES), lambda j: (0, 0)),
        ],
        out_specs=[
            pl.BlockSpec((_NBLK, NUM_REL), lambda j: (j, 0)),
            pl.BlockSpec((_NBLK, HIDDEN), lambda j: (j, 0)),
            pl.BlockSpec((NUM_REL, _NBLK, HIDDEN), lambda j: (0, j, 0)),
        ],
        out_shape=[
            jax.ShapeDtypeStruct((N, NUM_REL), jnp.float32),
            jax.ShapeDtypeStruct((N, HIDDEN), jnp.float32),
            jax.ShapeDtypeStruct((NUM_REL, N, HIDDEN), jnp.float32),
        ],
    )(deg_parts, wb0f, wb0id, w_rel0)


def _mid_body(h0_ref, bias_ref, wr1_ref, wb1_ref, t1_ref):
    h = jnp.maximum(h0_ref[...] + bias_ref[...], 0.0)
    wr1 = wr1_ref[...]
    for r in range(NUM_REL):
        w = wr1[r, 0] * wb1_ref[0]
        for b in range(1, NUM_BASES):
            w = w + wr1[r, b] * wb1_ref[b]
        t1_ref[r] = jnp.dot(h, w, preferred_element_type=jnp.float32,
                            precision=_HI)


def _mid(h0, bias, w_rel1, w_bases1):
    grid = (N // _NBLK,)
    return pl.pallas_call(
        _mid_body,
        grid=grid,
        in_specs=[
            pl.BlockSpec((_NBLK, HIDDEN), lambda j: (j, 0)),
            pl.BlockSpec((_NBLK, HIDDEN), lambda j: (j, 0)),
            pl.BlockSpec((NUM_REL, NUM_BASES), lambda j: (0, 0)),
            pl.BlockSpec((NUM_BASES, HIDDEN, OUT), lambda j: (0, 0, 0)),
        ],
        out_specs=pl.BlockSpec((NUM_REL, _NBLK, OUT), lambda j: (0, j, 0)),
        out_shape=jax.ShapeDtypeStruct((NUM_REL, N, OUT), jnp.float32),
    )(h0, bias, w_rel1, w_bases1)


def _fin_body(a_ref, b_ref, y_ref):
    y_ref[0] = a_ref[...] + b_ref[...]


def _fin(a, b):
    grid = (N // _NBLK,)
    return pl.pallas_call(
        _fin_body,
        grid=grid,
        in_specs=[
            pl.BlockSpec((_NBLK, OUT), lambda j: (j, 0)),
            pl.BlockSpec((_NBLK, OUT), lambda j: (j, 0)),
        ],
        out_specs=pl.BlockSpec((1, _NBLK, OUT), lambda j: (0, j, 0)),
        out_shape=jax.ShapeDtypeStruct((1, N, OUT), jnp.float32),
    )(a, b)


def kernel(X, edge_index, edge_type, w_bases0, w_rel0, w_bases1, w_rel1):
    src, dst = edge_index[0], edge_index[1]
    pad = EP - E
    srcp = jnp.concatenate([src, jnp.zeros((pad,), jnp.int32)])
    dstp = jnp.concatenate([dst, jnp.full((pad,), N, jnp.int32)])
    tp = jnp.concatenate([edge_type, jnp.zeros((pad,), jnp.int32)])

    degp = _deg_counts(dstp, tp)
    deg_parts = degp.reshape(NW, DEG_WORDS)[:, : 4 * N].reshape(NW, N, NUM_REL)

    recip, bias, t0 = _prep(deg_parts, w_bases0, w_rel0)
    recip_flat = recip.reshape(4 * N)
    t0_flat = t0.reshape(NUM_REL * N, HIDDEN)

    gip, dip, cnts = _partition(srcp, dstp, tp)

    h0pa = _edge_pass(gip, dip, cnts, t0_flat[:, :HW], recip_flat)
    h0pb = _edge_pass(gip, dip, cnts, t0_flat[:, HW:], recip_flat)
    h0 = jnp.concatenate(
        [h0pa[:, :HALF, :].reshape(N, HW), h0pb[:, :HALF, :].reshape(N, HW)],
        axis=1)
    t1 = _mid(h0, bias, w_rel1, w_bases1)
    t1_flat = t1.reshape(NUM_REL * N, OUT)

    outpa = _edge_pass(gip, dip, cnts, t1_flat[:, :HW], recip_flat)
    outpb = _edge_pass(gip, dip, cnts, t1_flat[:, HW:], recip_flat)
    y = jnp.concatenate(
        [outpa[:, :HALF, :].reshape(N, HW), outpb[:, :HALF, :].reshape(N, HW)],
        axis=1).reshape(1, N, OUT)

    p = jnp.ones((1, N), jnp.float32)
    lam = jnp.ones((1, N), jnp.float32)
    return (y, p, lam)


# speedup vs baseline: 1.0316x; 1.0316x over previous
import functools

import jax
import jax.numpy as jnp
from jax import lax
from jax.experimental import pallas as pl
from jax.experimental.pallas import tpu as pltpu
from jax.experimental.pallas import tpu_sc as plsc

N = 10000
E = 640000
NUM_REL = 4
NUM_BASES = 2
HIDDEN = 64
OUT = 64

NW = 32
TPW = 20480
EP = NW * TPW
B = 1024
NCH = TPW // B
NG = B // 16
KD = B // 128

ACC0_ROWS = 10112
DEG_ROWS = 40960
DEG_STRIPE = DEG_ROWS // 16
A0_STRIPE = ACC0_ROWS // 16

_NBLK = 1000

_HI = jax.lax.Precision.HIGHEST
_mesh = plsc.VectorSubcoreMesh(core_axis_name="c", subcore_axis_name="s")
_SC_PARAMS = pltpu.CompilerParams(needs_layout_passes=False,
                                  use_tc_tiling_on_sc=False)


DEG_WORDS = 40016


def _deg_body(dst_hbm, t_hbm, outd, dstv, tv, degv):
    c = lax.axis_index("c")
    s = lax.axis_index("s")
    wid = c * 16 + s

    lanes = lax.iota(jnp.int32, 16)
    zero16 = jnp.zeros((16,), jnp.float32)
    ones16 = jnp.ones((16,), jnp.float32)

    def zdeg(g, _):
        degv[pl.ds(g * 16, 16)] = zero16
        return 0

    lax.fori_loop(0, DEG_WORDS // 16, zdeg, 0)

    def chunk(ci, _):
        base = wid * TPW + ci * B
        pltpu.sync_copy(dst_hbm.at[pl.ds(base, B)], dstv)
        pltpu.sync_copy(t_hbm.at[pl.ds(base, B)], tv)

        def grp(g, _):
            d16 = dstv[pl.ds(g * 16, 16)]
            t16 = tv[pl.ds(g * 16, 16)]
            gi = d16 * 4 + t16
            for j in range(16):
                plsc.addupdate_scatter(degv, [gi], ones16, mask=lanes == j)
            return 0

        lax.fori_loop(0, NG, grp, 0)
        return 0

    lax.fori_loop(0, NCH, chunk, 0)
    pltpu.sync_copy(degv, outd.at[pl.ds(wid * DEG_WORDS, DEG_WORDS)])


def _deg_counts(dstp, tp):
    f = pl.kernel(
        _deg_body,
        out_type=jax.ShapeDtypeStruct((NW * DEG_WORDS,), jnp.float32),
        mesh=_mesh,
        compiler_params=_SC_PARAMS,
        scratch_types=[
            pltpu.VMEM((B,), jnp.int32),
            pltpu.VMEM((B,), jnp.int32),
            pltpu.VMEM((DEG_WORDS,), jnp.float32),
        ],
    )
    return f(dstp, tp)


HALF = N // 2
ACC_ROWS = 5120
HW = 32
CAP = TPW + 16


def _part_body(src_hbm, dst_hbm, t_hbm, gi_out, di_out, cnt_out,
               srcv, dstv, tv, giA, diA, giB, diB, cbuf, sem_l):
    c = lax.axis_index("c")
    s = lax.axis_index("s")
    wid = c * 16 + s
    lanes = lax.iota(jnp.int32, 16)

    zero16 = jnp.zeros((16,), jnp.int32)
    padA = zero16 + 4 * HALF
    padB = zero16 + 4 * N

    def pre(g, _):
        giA[pl.ds(g * 16, 16)] = zero16
        diA[pl.ds(g * 16, 16)] = padA
        giB[pl.ds(g * 16, 16)] = zero16
        diB[pl.ds(g * 16, 16)] = padB
        return 0

    lax.fori_loop(0, CAP // 16, pre, 0)

    def chunk(ci, offs):
        base = wid * TPW + ci * B
        dls = [pltpu.async_copy(src_hbm.at[pl.ds(base, B)], srcv, sem_l),
               pltpu.async_copy(dst_hbm.at[pl.ds(base, B)], dstv, sem_l),
               pltpu.async_copy(t_hbm.at[pl.ds(base, B)], tv, sem_l)]
        for d in dls:
            d.wait()

        def grp(g, offs2):
            offa, offb = offs2
            s16 = srcv[pl.ds(g * 16, 16)]
            d16 = dstv[pl.ds(g * 16, 16)]
            t16 = tv[pl.ds(g * 16, 16)]
            gi = t16 * N + s16
            di = d16 * 4 + t16
            ma = d16 < HALF
            mb = (d16 >= HALF) & (d16 < N)
            plsc.store_compressed(giA.at[pl.ds(offa, 16)], gi, mask=ma)
            plsc.store_compressed(diA.at[pl.ds(offa, 16)], di, mask=ma)
            plsc.store_compressed(giB.at[pl.ds(offb, 16)], gi, mask=mb)
            plsc.store_compressed(diB.at[pl.ds(offb, 16)], di, mask=mb)
            na = jnp.sum(ma.astype(jnp.int32))
            nb = jnp.sum(mb.astype(jnp.int32))
            return (offa + na, offb + nb)

        return lax.fori_loop(0, NG, grp, offs)

    offa, offb = lax.fori_loop(0, NCH, chunk, (0, 0))

    cbuf[pl.ds(0, 16)] = jnp.where(lanes == 0, offa, 0)
    pltpu.sync_copy(cbuf, cnt_out.at[pl.ds(wid * 16, 16)])
    cbuf[pl.ds(0, 16)] = jnp.where(lanes == 0, offb, 0)
    pltpu.sync_copy(cbuf, cnt_out.at[pl.ds((32 + wid) * 16, 16)])

    pltpu.sync_copy(giA.at[pl.ds(0, TPW)], gi_out.at[pl.ds(wid * TPW, TPW)])
    pltpu.sync_copy(diA.at[pl.ds(0, TPW)], di_out.at[pl.ds(wid * TPW, TPW)])
    pltpu.sync_copy(giB.at[pl.ds(0, TPW)],
                    gi_out.at[pl.ds((32 + wid) * TPW, TPW)])
    pltpu.sync_copy(diB.at[pl.ds(0, TPW)],
                    di_out.at[pl.ds((32 + wid) * TPW, TPW)])


def _partition(srcp, dstp, tp):
    f = pl.kernel(
        _part_body,
        out_type=(
            jax.ShapeDtypeStruct((64 * TPW,), jnp.int32),
            jax.ShapeDtypeStruct((64 * TPW,), jnp.int32),
            jax.ShapeDtypeStruct((64 * 16,), jnp.int32),
        ),
        mesh=_mesh,
        compiler_params=_SC_PARAMS,
        scratch_types=[
            pltpu.VMEM((B,), jnp.int32),
            pltpu.VMEM((B,), jnp.int32),
            pltpu.VMEM((B,), jnp.int32),
            pltpu.VMEM((CAP,), jnp.int32),
            pltpu.VMEM((CAP,), jnp.int32),
            pltpu.VMEM((CAP,), jnp.int32),
            pltpu.VMEM((CAP,), jnp.int32),
            pltpu.VMEM((16,), jnp.int32),
            pltpu.SemaphoreType.DMA,
        ],
    )
    return f(srcp, dstp, tp)


def _edge_body(gi_hbm, di_hbm, cnt_hbm, table_hbm, recip_hbm, outp,
               gilv, dilv, sidx, coefv, rows, recipv, cbufv, acc,
               sem_l, *sems):
    sem_g, sem_s = sems[:KD], sems[KD:]
    c = lax.axis_index("c")
    s = lax.axis_index("s")
    lanes = lax.iota(jnp.int32, 16)
    lo = c * HALF

    pltpu.sync_copy(recip_hbm, recipv.at[pl.ds(0, 4 * N)])
    recipv[pl.ds(4 * N, 16)] = jnp.zeros((16,), jnp.float32)

    zero16 = jnp.zeros((16,), jnp.float32)

    def zrow(i, _):
        for k in range(HW // 16):
            rows[i, pl.ds(k * 16, 16)] = zero16
        return 0

    lax.fori_loop(0, B, zrow, 0)
    pltpu.sync_copy(rows.at[pl.ds(0, 320)], acc.at[pl.ds(s * 320, 320)])
    plsc.subcore_barrier()

    for k in range(2):
        q = c * 32 + 2 * s + k
        pltpu.sync_copy(cnt_hbm.at[pl.ds(q * 16, 16)], cbufv)
        cv = cbufv[pl.ds(0, 16)]
        cnt = jnp.sum(jnp.where(lanes == 0, cv, 0))
        nch = (cnt + B - 1) // B

        dpre = [pltpu.async_copy(gi_hbm.at[pl.ds(q * TPW, TPW)], gilv, sem_l),
                pltpu.async_copy(di_hbm.at[pl.ds(q * TPW, TPW)], dilv, sem_l)]
        for d in dpre:
            d.wait()

        def chunk(ci, _):
            cb = ci * B

            @plsc.parallel_loop(0, NG, 1, unroll=2)
            def grp(g):
                di16 = dilv[pl.ds(cb + g * 16, 16)]
                u16 = (di16 >> 2) - lo
                sidx[g // 8, pl.ds((g % 8) * 16, 16)] = jnp.where(
                    u16 >= HALF, HALF + lanes, u16)
                coefv[pl.ds(g * 16, 16)] = plsc.load_gather(recipv, [di16])

            dgs = [pltpu.async_copy(
                       table_hbm.at[gilv.at[pl.ds(cb + j * 128, 128)]],
                       rows.at[pl.ds(j * 128, 128)], sem_g[j])
                   for j in range(KD)]
            for d in dgs:
                d.wait()

            @plsc.parallel_loop(0, NG, 1, unroll=2)
            def scale(g):
                for jj in range(16):
                    e = g * 16 + jj
                    cof = plsc.load_gather(
                        coefv, [jnp.zeros((16,), jnp.int32) + e])
                    for kk in range(HW // 16):
                        rows[e, pl.ds(kk * 16, 16)] = (
                            rows[e, pl.ds(kk * 16, 16)] * cof)

            dss = [pltpu.async_copy(rows.at[pl.ds(j * 128, 128)],
                                    acc.at[sidx.at[j]], sem_s[j], add=True)
                   for j in range(KD)]
            for d in dss:
                d.wait()
            return 0

        lax.fori_loop(0, nch, chunk, 0)

    plsc.subcore_barrier()

    pltpu.sync_copy(acc.at[pl.ds(s * 320, 320)],
                    outp.at[c, pl.ds(s * 320, 320)])


def _edge_pass(gip, dip, cnts, table, recip_flat):
    f = pl.kernel(
        _edge_body,
        out_type=jax.ShapeDtypeStruct((2, ACC_ROWS, HW), jnp.float32),
        mesh=_mesh,
        compiler_params=_SC_PARAMS,
        scratch_types=[
            pltpu.VMEM((TPW,), jnp.int32),
            pltpu.VMEM((TPW,), jnp.int32),
            pltpu.VMEM((KD, 128), jnp.int32),
            pltpu.VMEM((B,), jnp.float32),
            pltpu.VMEM((B, HW), jnp.float32),
            pltpu.VMEM((4 * N + 16,), jnp.float32),
            pltpu.VMEM((16,), jnp.int32),
            pltpu.VMEM_SHARED((ACC_ROWS, HW), jnp.float32),
            pltpu.SemaphoreType.DMA,
        ] + [pltpu.SemaphoreType.DMA] * (2 * KD),
    )
    return f(gip, dip, cnts, table, recip_flat)


def _prep_body(parts_ref, wb0f_ref, wb0id_ref, wr0_ref,
               recip_ref, bias_ref, t0_ref):
    d = jnp.sum(parts_ref[...], axis=0)
    recip_ref[...] = jnp.where(d > 0, 1.0 / jnp.maximum(d, 1.0), 0.0)
    ind = (d > 0).astype(jnp.float32)
    wr0 = wr0_ref[...]
    colsum = jnp.sum(wb0f_ref[...], axis=1)
    sm = jnp.dot(wr0, colsum, precision=_HI)
    bias_ref[...] = jnp.dot(ind, sm, precision=_HI)
    for r in range(NUM_REL):
        accv = wr0[r, 0] * wb0id_ref[0]
        for b in range(1, NUM_BASES):
            accv = accv + wr0[r, b] * wb0id_ref[b]
        t0_ref[r] = accv


def _prep(deg_parts, w_bases0, w_rel0):
    wb0f = w_bases0[:, :OUT, :]
    wb0id = w_bases0[:, OUT:, :]
    grid = (N // _NBLK,)
    return pl.pallas_call(
        _prep_body,
        grid=grid,
        in_specs=[
            pl.BlockSpec((NW, _NBLK, NUM_REL), lambda j: (0, j, 0)),
            pl.BlockSpec((NUM_BASES, OUT, HIDDEN), lambda j: (0, 0, 0)),
            pl.BlockSpec((NUM_BASES, _NBLK, HIDDEN), lambda j: (0, j, 0)),
            pl.BlockSpec((NUM_REL, NUM_BASES), lambda j: (0, 0)),
        ],
        out_specs=[
            pl.BlockSpec((_NBLK, NUM_REL), lambda j: (j, 0)),
            pl.BlockSpec((_NBLK, HIDDEN), lambda j: (j, 0)),
            pl.BlockSpec((NUM_REL, _NBLK, HIDDEN), lambda j: (0, j, 0)),
        ],
        out_shape=[
            jax.ShapeDtypeStruct((N, NUM_REL), jnp.float32),
            jax.ShapeDtypeStruct((N, HIDDEN), jnp.float32),
            jax.ShapeDtypeStruct((NUM_REL, N, HIDDEN), jnp.float32),
        ],
    )(deg_parts, wb0f, wb0id, w_rel0)


def _mid_body(h0_ref, bias_ref, wr1_ref, wb1_ref, t1_ref):
    h = jnp.maximum(h0_ref[...] + bias_ref[...], 0.0)
    wr1 = wr1_ref[...]
    for r in range(NUM_REL):
        w = wr1[r, 0] * wb1_ref[0]
        for b in range(1, NUM_BASES):
            w = w + wr1[r, b] * wb1_ref[b]
        t1_ref[r] = jnp.dot(h, w, preferred_element_type=jnp.float32,
                            precision=_HI)


def _mid(h0, bias, w_rel1, w_bases1):
    grid = (N // _NBLK,)
    return pl.pallas_call(
        _mid_body,
        grid=grid,
        in_specs=[
            pl.BlockSpec((_NBLK, HIDDEN), lambda j: (j, 0)),
            pl.BlockSpec((_NBLK, HIDDEN), lambda j: (j, 0)),
            pl.BlockSpec((NUM_REL, NUM_BASES), lambda j: (0, 0)),
            pl.BlockSpec((NUM_BASES, HIDDEN, OUT), lambda j: (0, 0, 0)),
        ],
        out_specs=pl.BlockSpec((NUM_REL, _NBLK, OUT), lambda j: (0, j, 0)),
        out_shape=jax.ShapeDtypeStruct((NUM_REL, N, OUT), jnp.float32),
    )(h0, bias, w_rel1, w_bases1)


def _fin_body(a_ref, b_ref, y_ref):
    y_ref[0] = a_ref[...] + b_ref[...]


def _fin(a, b):
    grid = (N // _NBLK,)
    return pl.pallas_call(
        _fin_body,
        grid=grid,
        in_specs=[
            pl.BlockSpec((_NBLK, OUT), lambda j: (j, 0)),
            pl.BlockSpec((_NBLK, OUT), lambda j: (j, 0)),
        ],
        out_specs=pl.BlockSpec((1, _NBLK, OUT), lambda j: (0, j, 0)),
        out_shape=jax.ShapeDtypeStruct((1, N, OUT), jnp.float32),
    )(a, b)


def kernel(X, edge_index, edge_type, w_bases0, w_rel0, w_bases1, w_rel1):
    src, dst = edge_index[0], edge_index[1]
    pad = EP - E
    srcp = jnp.concatenate([src, jnp.zeros((pad,), jnp.int32)])
    dstp = jnp.concatenate([dst, jnp.full((pad,), N, jnp.int32)])
    tp = jnp.concatenate([edge_type, jnp.zeros((pad,), jnp.int32)])

    degp = _deg_counts(dstp, tp)
    deg_parts = degp.reshape(NW, DEG_WORDS)[:, : 4 * N].reshape(NW, N, NUM_REL)

    recip, bias, t0 = _prep(deg_parts, w_bases0, w_rel0)
    recip_flat = recip.reshape(4 * N)
    t0_flat = t0.reshape(NUM_REL * N, HIDDEN)

    gip, dip, cnts = _partition(srcp, dstp, tp)

    h0pa = _edge_pass(gip, dip, cnts, t0_flat[:, :HW], recip_flat)
    h0pb = _edge_pass(gip, dip, cnts, t0_flat[:, HW:], recip_flat)
    h0 = jnp.concatenate(
        [h0pa[:, :HALF, :].reshape(N, HW), h0pb[:, :HALF, :].reshape(N, HW)],
        axis=1)
    t1 = _mid(h0, bias, w_rel1, w_bases1)
    t1_flat = t1.reshape(NUM_REL * N, OUT)

    outpa = _edge_pass(gip, dip, cnts, t1_flat[:, :HW], recip_flat)
    outpb = _edge_pass(gip, dip, cnts, t1_flat[:, HW:], recip_flat)
    y = jnp.concatenate(
        [outpa[:, :HALF, :].reshape(N, HW), outpb[:, :HALF, :].reshape(N, HW)],
        axis=1).reshape(1, N, OUT)

    p = jnp.ones((1, N), jnp.float32)
    lam = jnp.ones((1, N), jnp.float32)
    return (y, p, lam)
